# baseline (speedup 1.0000x reference)
import functools

import jax
import jax.numpy as jnp
from jax import lax
from jax.experimental import pallas as pl
from jax.experimental.pallas import tpu as pltpu
from jax.experimental.pallas import tpu_sc as plsc

N_USERS = 25000
N_NODES = 50000
N_EDGES = 800000
DIM = 64
BATCH = 4096

HALF = N_NODES // 2
ACC_ROWS = 25600
EC = 80
SB = 2000
CPS = SB // EC
ZCH = 40
PC = BATCH // 32

E_PAD = 800256
PPT = E_PAD // 32
SB2 = 8336
NB2 = PPT // SB2
SLOT = 26000
BUFP = 26016

_MESH = plsc.VectorSubcoreMesh(
    core_axis_name="c", subcore_axis_name="s", num_cores=2, num_subcores=16
)
_PARAMS = pltpu.CompilerParams(
    needs_layout_passes=False, use_tc_tiling_on_sc=False)

_F32 = jnp.float32
_I32 = jnp.int32


def _part_body(srcg, dstg, valg, psrc, pdst, pval, pcnt,
               sidx, sdst, sval, bsrc, bdst, bval, cnt_v):
    c = lax.axis_index("c")
    s = lax.axis_index("s")
    w = s * 2 + c
    ebase = w * PPT
    zi = jnp.zeros((16,), _I32)
    zf = jnp.zeros((16,), _F32)

    for h in (0, 1):
        def _blk(b, off):
            bb = ebase + b * SB2
            pltpu.sync_copy(srcg.at[pl.ds(bb, SB2)], sidx)
            pltpu.sync_copy(dstg.at[pl.ds(bb, SB2)], sdst)
            pltpu.sync_copy(valg.at[pl.ds(bb, SB2)], sval)

            def _grp(g, off):
                svec = sidx[pl.ds(g * 16, 16)]
                dvec = sdst[pl.ds(g * 16, 16)]
                vvec = sval[pl.ds(g * 16, 16)]
                if h == 0:
                    m = dvec < HALF
                    ldv = dvec
                else:
                    m = dvec >= HALF
                    ldv = dvec - HALF
                plsc.store_compressed(bsrc.at[pl.ds(off, 16)], svec, mask=m)
                plsc.store_compressed(bdst.at[pl.ds(off, 16)], ldv, mask=m)
                plsc.store_compressed(bval.at[pl.ds(off, 16)], vvec, mask=m)
                return off + jnp.sum(m.astype(_I32))

            return lax.fori_loop(0, SB2 // 16, _grp, off)

        off = lax.fori_loop(0, NB2, _blk, 0)
        pe = 14000
        nz = (pe - off + 15) >> 4

        def _zf(k, _):
            bsrc[pl.ds(off + k * 16, 16)] = zi
            bdst[pl.ds(off + k * 16, 16)] = zi
            bval[pl.ds(off + k * 16, 16)] = zf
            return 0

        lax.fori_loop(0, nz, _zf, 0)
        slotbase = (h * 32 + w) * SLOT
        pltpu.sync_copy(bsrc.at[pl.ds(0, SLOT)],
                        psrc.at[pl.ds(slotbase, SLOT)])
        pltpu.sync_copy(bdst.at[pl.ds(0, SLOT)],
                        pdst.at[pl.ds(slotbase, SLOT)])
        pltpu.sync_copy(bval.at[pl.ds(0, SLOT)],
                        pval.at[pl.ds(slotbase, SLOT)])
        cnt_v[pl.ds(0, 16)] = zi + pe
        pltpu.sync_copy(cnt_v, pcnt.at[pl.ds((h * 32 + w) * 16, 16)])


_part = functools.partial(
    pl.kernel,
    out_type=(
        jax.ShapeDtypeStruct((2 * 32 * SLOT,), _I32),
        jax.ShapeDtypeStruct((2 * 32 * SLOT,), _I32),
        jax.ShapeDtypeStruct((2 * 32 * SLOT,), _F32),
        jax.ShapeDtypeStruct((2 * 32 * 16,), _I32),
    ),
    mesh=_MESH,
    compiler_params=_PARAMS,
    scratch_types=[
        pltpu.VMEM((SB2,), _I32),
        pltpu.VMEM((SB2,), _I32),
        pltpu.VMEM((SB2,), _F32),
        pltpu.VMEM((BUFP,), _I32),
        pltpu.VMEM((BUFP,), _I32),
        pltpu.VMEM((BUFP,), _F32),
        pltpu.VMEM((16,), _I32),
    ],
)(_part_body)


def _scale_chunk(rows_v, ldst_v, sdst, sval, off0):
    for j in range(EC // 16):
        off = off0 + j * 16
        ldst_v[pl.ds(j * 16, 16)] = sdst[pl.ds(off, 16)]
        for e in range(16):
            bc = plsc.load_gather(sval, [jnp.full((16,), off + e, _I32)])
            row = j * 16 + e
            for d in range(4):
                sl = rows_v[row, pl.ds(d * 16, 16)]
                rows_v[row, pl.ds(d * 16, 16)] = sl * bc


def _layer_body(ecur, psrc, pdst, pval, pcnt, enext, acc,
                sidx, sdst, sval, cnt_v, ldst0, ldst1, rows0, rows1,
                stage, sem0, sem1, ssem0, ssem1):
    c = lax.axis_index("c")
    s = lax.axis_index("s")
    base_node = c * HALF
    zeros16 = jnp.zeros((16,), _F32)

    def _zrow(r, _):
        for d in range(4):
            stage[r, pl.ds(d * 16, 16)] = zeros16
        return 0

    lax.fori_loop(0, ZCH, _zrow, 0)
    tz = s * (ACC_ROWS // 16)

    def _zacc(i, _):
        pltpu.sync_copy(stage, acc.at[pl.ds(tz + i * ZCH, ZCH)])
        return 0

    with jax.named_scope("zerophase"):
        lax.fori_loop(0, ACC_ROWS // 16 // ZCH, _zacc, 0)
    plsc.subcore_barrier()

    def _gather(ch, rows_v, sem):
        pltpu.async_copy(ecur.at[sidx.at[pl.ds(ch * EC, EC)]], rows_v, sem)

    def _wait_g(rows_v, sem):
        pltpu.make_async_copy(ecur.at[sidx.at[pl.ds(0, EC)]], rows_v,
                              sem).wait()

    def _chunk(ch, rows_v, ldst_v, ssem):
        _scale_chunk(rows_v, ldst_v, sdst, sval, ch * EC)
        pltpu.sync_copy(rows_v, acc.at[ldst_v], add=True)

    def _slot(q, _):
        p = 2 * s + q
        slotbase = (c * 32 + p) * SLOT
        pltpu.sync_copy(pcnt.at[pl.ds((c * 32 + p) * 16, 16)], cnt_v)
        cnt = jnp.sum(cnt_v[pl.ds(0, 16)]) >> 4
        nsb = 7

        def _sb(sb, _):
            base = slotbase + sb * SB
            with jax.named_scope("stage"):
                pltpu.sync_copy(psrc.at[pl.ds(base, SB)], sidx)
                pltpu.sync_copy(pdst.at[pl.ds(base, SB)], sdst)
                pltpu.sync_copy(pval.at[pl.ds(base, SB)], sval)
            with jax.named_scope("pipe"):
                _gather(0, rows0, sem0)

                def _pair(m, _):
                    ch0 = 2 * m
                    _gather(ch0 + 1, rows1, sem1)
                    _wait_g(rows0, sem0)
                    _chunk(ch0, rows0, ldst0, ssem0)
                    _gather(ch0 + 2, rows0, sem0)
                    _wait_g(rows1, sem1)
                    _chunk(ch0 + 1, rows1, ldst1, ssem1)
                    return 0

                lax.fori_loop(0, (CPS - 1) // 2, _pair, 0)
                _wait_g(rows0, sem0)
                _chunk(CPS - 1, rows0, ldst0, ssem0)
            return 0

        lax.fori_loop(0, nsb, _sb, 0)
        return 0

    with jax.named_scope("edges"):
        lax.fori_loop(0, 2, _slot, 0)
    plsc.subcore_barrier()

    nch = jnp.where(s == 0, 40, 39)

    def _cp(k, _):
        r0 = (s + k * 16) * ZCH
        pltpu.sync_copy(acc.at[pl.ds(r0, ZCH)],
                        enext.at[pl.ds(base_node + r0, ZCH)])
        return 0

    lax.fori_loop(0, nch, _cp, 0)


_layer = functools.partial(
    pl.kernel,
    out_type=jax.ShapeDtypeStruct((N_NODES, DIM), _F32),
    mesh=_MESH,
    compiler_params=_PARAMS,
    scratch_types=[
        pltpu.VMEM_SHARED((ACC_ROWS, DIM), _F32),
        pltpu.VMEM((SB,), _I32),
        pltpu.VMEM((SB,), _I32),
        pltpu.VMEM((SB,), _F32),
        pltpu.VMEM((16,), _I32),
        pltpu.VMEM((EC,), _I32),
        pltpu.VMEM((EC,), _I32),
        pltpu.VMEM((EC, DIM), _F32),
        pltpu.VMEM((EC, DIM), _F32),
        pltpu.VMEM((ZCH, DIM), _F32),
        pltpu.SemaphoreType.DMA,
        pltpu.SemaphoreType.DMA,
        pltpu.SemaphoreType.DMA,
        pltpu.SemaphoreType.DMA,
    ],
)(_layer_body)


def _score_body(uidx, iidx, e0, l1, l2, l3, out, uv, iv, t0, t1, t2, t3,
                usum, isum, sc_v, sem):
    c = lax.axis_index("c")
    s = lax.axis_index("s")
    wid = s * 2 + c
    base = wid * PC
    pltpu.sync_copy(uidx.at[pl.ds(base, PC)], uv)
    pltpu.sync_copy(iidx.at[pl.ds(base, PC)], iv)

    def _gather4(idx_v, dst_sum):
        pltpu.async_copy(e0.at[idx_v], t0, sem).wait()
        pltpu.async_copy(l1.at[idx_v], t1, sem).wait()
        pltpu.async_copy(l2.at[idx_v], t2, sem).wait()
        pltpu.async_copy(l3.at[idx_v], t3, sem).wait()

        def _sumr(r, _):
            for d in range(4):
                sl = pl.ds(d * 16, 16)
                dst_sum[r, sl] = (t0[r, sl] + t1[r, sl] + t2[r, sl]
                                  + t3[r, sl])
            return 0

        lax.fori_loop(0, PC, _sumr, 0)

    _gather4(uv, usum)
    _gather4(iv, isum)

    lanes = jnp.arange(16, dtype=_I32)
    for g in range(PC // 16):
        rowsel = g * 16 + lanes
        acc = jnp.zeros((16,), _F32)
        for d in range(DIM):
            col = jnp.full((16,), d, _I32)
            ua = plsc.load_gather(usum, [rowsel, col])
            ia = plsc.load_gather(isum, [rowsel, col])
            acc = acc + ua * ia
        sc_v[pl.ds(g * 16, 16)] = acc * 0.0625

    pltpu.sync_copy(sc_v, out.at[pl.ds(base, PC)])


_score = functools.partial(
    pl.kernel,
    out_type=jax.ShapeDtypeStruct((BATCH,), _F32),
    mesh=_MESH,
    compiler_params=_PARAMS,
    scratch_types=[
        pltpu.VMEM((PC,), _I32),
        pltpu.VMEM((PC,), _I32),
        pltpu.VMEM((PC, DIM), _F32),
        pltpu.VMEM((PC, DIM), _F32),
        pltpu.VMEM((PC, DIM), _F32),
        pltpu.VMEM((PC, DIM), _F32),
        pltpu.VMEM((PC, DIM), _F32),
        pltpu.VMEM((PC, DIM), _F32),
        pltpu.VMEM((PC,), _F32),
        pltpu.SemaphoreType.DMA,
    ],
)(_score_body)


def kernel(users, items, adj_indices, adj_values, user_emb, item_emb):
    e0 = jnp.concatenate([user_emb, item_emb], axis=0)
    pad = E_PAD - N_EDGES
    src = jnp.pad(adj_indices[1].astype(_I32), (0, pad))
    dst = jnp.pad(adj_indices[0].astype(_I32), (0, pad))
    val = jnp.pad(adj_values, (0, pad))
    psrc, pdst, pval, pcnt = _part(src, dst, val)
    l1 = _layer(e0, psrc, pdst, pval, pcnt)
    l2 = _layer(l1, psrc, pdst, pval, pcnt)
    l3 = _layer(l2, psrc, pdst, pval, pcnt)
    return _score(users.astype(_I32), (items + N_USERS).astype(_I32),
                  e0, l1, l2, l3)


# speedup vs baseline: 1.1100x; 1.1100x over previous
import functools

import jax
import jax.numpy as jnp
from jax import lax
from jax.experimental import pallas as pl
from jax.experimental.pallas import tpu as pltpu
from jax.experimental.pallas import tpu_sc as plsc

N_USERS = 25000
N_NODES = 50000
N_EDGES = 800000
DIM = 64
BATCH = 4096

HALF = N_NODES // 2
ACC_ROWS = 25600
EC = 80
SB = 2000
CPS = SB // EC
ZCH = 40
PC = BATCH // 32

E_PAD = 800256
PPT = E_PAD // 32
SB2 = 8336
NB2 = PPT // SB2
SLOT = 26000
BUFP = 26016

_MESH = plsc.VectorSubcoreMesh(
    core_axis_name="c", subcore_axis_name="s", num_cores=2, num_subcores=16
)
_PARAMS = pltpu.CompilerParams(
    needs_layout_passes=False, use_tc_tiling_on_sc=False)

_F32 = jnp.float32
_I32 = jnp.int32


def _part_body(srcg, dstg, valg, psrc, pdst, pval, pcnt,
               sidx, sdst, sval, bsrc, bdst, bval, cnt_v):
    c = lax.axis_index("c")
    s = lax.axis_index("s")
    w = s * 2 + c
    ebase = w * PPT
    zi = jnp.zeros((16,), _I32)
    zf = jnp.zeros((16,), _F32)

    for h in (0, 1):
        def _blk(b, off):
            bb = ebase + b * SB2
            pltpu.sync_copy(srcg.at[pl.ds(bb, SB2)], sidx)
            pltpu.sync_copy(dstg.at[pl.ds(bb, SB2)], sdst)
            pltpu.sync_copy(valg.at[pl.ds(bb, SB2)], sval)

            def _grp(g, off):
                svec = sidx[pl.ds(g * 16, 16)]
                dvec = sdst[pl.ds(g * 16, 16)]
                vvec = sval[pl.ds(g * 16, 16)]
                if h == 0:
                    m = dvec < HALF
                    ldv = dvec
                else:
                    m = dvec >= HALF
                    ldv = dvec - HALF
                plsc.store_compressed(bsrc.at[pl.ds(off, 16)], svec, mask=m)
                plsc.store_compressed(bdst.at[pl.ds(off, 16)], ldv, mask=m)
                plsc.store_compressed(bval.at[pl.ds(off, 16)], vvec, mask=m)
                return off + jnp.sum(m.astype(_I32))

            return lax.fori_loop(0, SB2 // 16, _grp, off)

        off = lax.fori_loop(0, NB2, _blk, 0)
        pe = ((off + SB - 1) // SB) * SB
        nz = (pe - off + 15) >> 4

        def _zf(k, _):
            bsrc[pl.ds(off + k * 16, 16)] = zi
            bdst[pl.ds(off + k * 16, 16)] = zi
            bval[pl.ds(off + k * 16, 16)] = zf
            return 0

        lax.fori_loop(0, nz, _zf, 0)
        slotbase = (h * 32 + w) * SLOT
        pltpu.sync_copy(bsrc.at[pl.ds(0, SLOT)],
                        psrc.at[pl.ds(slotbase, SLOT)])
        pltpu.sync_copy(bdst.at[pl.ds(0, SLOT)],
                        pdst.at[pl.ds(slotbase, SLOT)])
        pltpu.sync_copy(bval.at[pl.ds(0, SLOT)],
                        pval.at[pl.ds(slotbase, SLOT)])
        cnt_v[pl.ds(0, 16)] = zi + pe
        pltpu.sync_copy(cnt_v, pcnt.at[pl.ds((h * 32 + w) * 16, 16)])


_part = functools.partial(
    pl.kernel,
    out_type=(
        jax.ShapeDtypeStruct((2 * 32 * SLOT,), _I32),
        jax.ShapeDtypeStruct((2 * 32 * SLOT,), _I32),
        jax.ShapeDtypeStruct((2 * 32 * SLOT,), _F32),
        jax.ShapeDtypeStruct((2 * 32 * 16,), _I32),
    ),
    mesh=_MESH,
    compiler_params=_PARAMS,
    scratch_types=[
        pltpu.VMEM((SB2,), _I32),
        pltpu.VMEM((SB2,), _I32),
        pltpu.VMEM((SB2,), _F32),
        pltpu.VMEM((BUFP,), _I32),
        pltpu.VMEM((BUFP,), _I32),
        pltpu.VMEM((BUFP,), _F32),
        pltpu.VMEM((16,), _I32),
    ],
)(_part_body)


def _scale_chunk(rows_v, ldst_v, sdst, sval, off0):
    for j in range(EC // 16):
        off = off0 + j * 16
        ldst_v[pl.ds(j * 16, 16)] = sdst[pl.ds(off, 16)]
        for e in range(16):
            bc = plsc.load_gather(sval, [jnp.full((16,), off + e, _I32)])
            row = j * 16 + e
            for d in range(4):
                sl = rows_v[row, pl.ds(d * 16, 16)]
                rows_v[row, pl.ds(d * 16, 16)] = sl * bc


def _layer_body(ecur, psrc, pdst, pval, pcnt, enext, acc,
                sidx, sdst, sval, cnt_v, ldst0, ldst1, rows0, rows1,
                stage, sem0, sem1, ssem0, ssem1):
    c = lax.axis_index("c")
    s = lax.axis_index("s")
    base_node = c * HALF
    zeros16 = jnp.zeros((16,), _F32)

    def _zrow(r, _):
        for d in range(4):
            stage[r, pl.ds(d * 16, 16)] = zeros16
        return 0

    lax.fori_loop(0, ZCH, _zrow, 0)
    tz = s * (ACC_ROWS // 16)

    def _zacc(i, _):
        pltpu.sync_copy(stage, acc.at[pl.ds(tz + i * ZCH, ZCH)])
        return 0

    with jax.named_scope("zerophase"):
        lax.fori_loop(0, ACC_ROWS // 16 // ZCH, _zacc, 0)
    plsc.subcore_barrier()

    def _gather(ch, rows_v, sem):
        pltpu.async_copy(ecur.at[sidx.at[pl.ds(ch * EC, EC)]], rows_v, sem)

    def _wait_g(rows_v, sem):
        with jax.named_scope("wg"):
            pltpu.make_async_copy(ecur.at[sidx.at[pl.ds(0, EC)]], rows_v,
                                  sem).wait()

    def _chunk(ch, rows_v, ldst_v, ssem):
        with jax.named_scope("comp"):
            _scale_chunk(rows_v, ldst_v, sdst, sval, ch * EC)
        with jax.named_scope("scat"):
            pltpu.sync_copy(rows_v, acc.at[ldst_v], add=True)

    def _slot(q, _):
        p = 2 * s + q
        slotbase = (c * 32 + p) * SLOT
        pltpu.sync_copy(pcnt.at[pl.ds((c * 32 + p) * 16, 16)], cnt_v)
        cnt = jnp.sum(cnt_v[pl.ds(0, 16)]) >> 4
        nsb = cnt // SB

        def _sb(sb, _):
            base = slotbase + sb * SB
            with jax.named_scope("stage"):
                pltpu.sync_copy(psrc.at[pl.ds(base, SB)], sidx)
                pltpu.sync_copy(pdst.at[pl.ds(base, SB)], sdst)
                pltpu.sync_copy(pval.at[pl.ds(base, SB)], sval)
            with jax.named_scope("pipe"):
                _gather(0, rows0, sem0)

                def _pair(m, _):
                    ch0 = 2 * m
                    _gather(ch0 + 1, rows1, sem1)
                    _wait_g(rows0, sem0)
                    _chunk(ch0, rows0, ldst0, ssem0)
                    _gather(ch0 + 2, rows0, sem0)
                    _wait_g(rows1, sem1)
                    _chunk(ch0 + 1, rows1, ldst1, ssem1)
                    return 0

                lax.fori_loop(0, (CPS - 1) // 2, _pair, 0)
                _wait_g(rows0, sem0)
                _chunk(CPS - 1, rows0, ldst0, ssem0)
            return 0

        lax.fori_loop(0, nsb, _sb, 0)
        return 0

    with jax.named_scope("edges"):
        lax.fori_loop(0, 2, _slot, 0)
    plsc.subcore_barrier()

    nch = jnp.where(s == 0, 40, 39)

    def _cp(k, _):
        r0 = (s + k * 16) * ZCH
        pltpu.sync_copy(acc.at[pl.ds(r0, ZCH)],
                        enext.at[pl.ds(base_node + r0, ZCH)])
        return 0

    lax.fori_loop(0, nch, _cp, 0)


_layer = functools.partial(
    pl.kernel,
    out_type=jax.ShapeDtypeStruct((N_NODES, DIM), _F32),
    mesh=_MESH,
    compiler_params=_PARAMS,
    scratch_types=[
        pltpu.VMEM_SHARED((ACC_ROWS, DIM), _F32),
        pltpu.VMEM((SB,), _I32),
        pltpu.VMEM((SB,), _I32),
        pltpu.VMEM((SB,), _F32),
        pltpu.VMEM((16,), _I32),
        pltpu.VMEM((EC,), _I32),
        pltpu.VMEM((EC,), _I32),
        pltpu.VMEM((EC, DIM), _F32),
        pltpu.VMEM((EC, DIM), _F32),
        pltpu.VMEM((ZCH, DIM), _F32),
        pltpu.SemaphoreType.DMA,
        pltpu.SemaphoreType.DMA,
        pltpu.SemaphoreType.DMA,
        pltpu.SemaphoreType.DMA,
    ],
)(_layer_body)


def _score_body(uidx, iidx, e0, l1, l2, l3, out, uv, iv, t0, t1, t2, t3,
                usum, isum, sc_v, sem):
    c = lax.axis_index("c")
    s = lax.axis_index("s")
    wid = s * 2 + c
    base = wid * PC
    pltpu.sync_copy(uidx.at[pl.ds(base, PC)], uv)
    pltpu.sync_copy(iidx.at[pl.ds(base, PC)], iv)

    def _gather4(idx_v, dst_sum):
        pltpu.async_copy(e0.at[idx_v], t0, sem).wait()
        pltpu.async_copy(l1.at[idx_v], t1, sem).wait()
        pltpu.async_copy(l2.at[idx_v], t2, sem).wait()
        pltpu.async_copy(l3.at[idx_v], t3, sem).wait()

        def _sumr(r, _):
            for d in range(4):
                sl = pl.ds(d * 16, 16)
                dst_sum[r, sl] = (t0[r, sl] + t1[r, sl] + t2[r, sl]
                                  + t3[r, sl])
            return 0

        lax.fori_loop(0, PC, _sumr, 0)

    _gather4(uv, usum)
    _gather4(iv, isum)

    lanes = jnp.arange(16, dtype=_I32)
    for g in range(PC // 16):
        rowsel = g * 16 + lanes
        acc = jnp.zeros((16,), _F32)
        for d in range(DIM):
            col = jnp.full((16,), d, _I32)
            ua = plsc.load_gather(usum, [rowsel, col])
            ia = plsc.load_gather(isum, [rowsel, col])
            acc = acc + ua * ia
        sc_v[pl.ds(g * 16, 16)] = acc * 0.0625

    pltpu.sync_copy(sc_v, out.at[pl.ds(base, PC)])


_score = functools.partial(
    pl.kernel,
    out_type=jax.ShapeDtypeStruct((BATCH,), _F32),
    mesh=_MESH,
    compiler_params=_PARAMS,
    scratch_types=[
        pltpu.VMEM((PC,), _I32),
        pltpu.VMEM((PC,), _I32),
        pltpu.VMEM((PC, DIM), _F32),
        pltpu.VMEM((PC, DIM), _F32),
        pltpu.VMEM((PC, DIM), _F32),
        pltpu.VMEM((PC, DIM), _F32),
        pltpu.VMEM((PC, DIM), _F32),
        pltpu.VMEM((PC, DIM), _F32),
        pltpu.VMEM((PC,), _F32),
        pltpu.SemaphoreType.DMA,
    ],
)(_score_body)


def kernel(users, items, adj_indices, adj_values, user_emb, item_emb):
    e0 = jnp.concatenate([user_emb, item_emb], axis=0)
    pad = E_PAD - N_EDGES
    src = jnp.pad(adj_indices[1].astype(_I32), (0, pad))
    dst = jnp.pad(adj_indices[0].astype(_I32), (0, pad))
    val = jnp.pad(adj_values, (0, pad))
    psrc, pdst, pval, pcnt = _part(src, dst, val)
    l1 = _layer(e0, psrc, pdst, pval, pcnt)
    l2 = _layer(l1, psrc, pdst, pval, pcnt)
    l3 = _layer(l2, psrc, pdst, pval, pcnt)
    return _score(users.astype(_I32), (items + N_USERS).astype(_I32),
                  e0, l1, l2, l3)


# speedup vs baseline: 2.5337x; 2.2827x over previous
import functools

import jax
import jax.numpy as jnp
from jax import lax
from jax.experimental import pallas as pl
from jax.experimental.pallas import tpu as pltpu
from jax.experimental.pallas import tpu_sc as plsc

N_USERS = 25000
N_NODES = 50000
N_EDGES = 800000
DIM = 64
BATCH = 4096

HALF = N_NODES // 2
ACC_ROWS = 25088
EC = 80
SB = 1600
CPS = SB // EC
ZZ = 56
ZCH = 40
PC = BATCH // 32

E_PAD = 800256
PPT = E_PAD // 32
SB2 = 8336
NB2 = PPT // SB2
SLOT = 25600
BUFP = 25616

_MESH = plsc.VectorSubcoreMesh(
    core_axis_name="c", subcore_axis_name="s", num_cores=2, num_subcores=16
)
_PARAMS = pltpu.CompilerParams(
    needs_layout_passes=False, use_tc_tiling_on_sc=False)

_F32 = jnp.float32
_I32 = jnp.int32


def _part_body(srcg, dstg, valg, psrc, pdst, pval, pcnt,
               sidx, sdst, sval, bsrc, bdst, bval, cnt_v):
    c = lax.axis_index("c")
    s = lax.axis_index("s")
    w = s * 2 + c
    ebase = w * PPT
    zi = jnp.zeros((16,), _I32)
    zf = jnp.zeros((16,), _F32)

    for h in (0, 1):
        def _blk(b, off):
            bb = ebase + b * SB2
            pltpu.sync_copy(srcg.at[pl.ds(bb, SB2)], sidx)
            pltpu.sync_copy(dstg.at[pl.ds(bb, SB2)], sdst)
            pltpu.sync_copy(valg.at[pl.ds(bb, SB2)], sval)

            def _grp(g, off):
                svec = sidx[pl.ds(g * 16, 16)]
                dvec = sdst[pl.ds(g * 16, 16)]
                vvec = sval[pl.ds(g * 16, 16)]
                if h == 0:
                    m = dvec < HALF
                    ldv = dvec
                else:
                    m = dvec >= HALF
                    ldv = dvec - HALF
                plsc.store_compressed(bsrc.at[pl.ds(off, 16)], svec, mask=m)
                plsc.store_compressed(bdst.at[pl.ds(off, 16)], ldv, mask=m)
                plsc.store_compressed(bval.at[pl.ds(off, 16)], vvec, mask=m)
                return off + jnp.sum(m.astype(_I32))

            return lax.fori_loop(0, SB2 // 16, _grp, off)

        off = lax.fori_loop(0, NB2, _blk, 0)
        pe = ((off + SB - 1) // SB) * SB
        nz = (pe - off + 15) >> 4

        def _zf(k, _):
            bsrc[pl.ds(off + k * 16, 16)] = zi
            bdst[pl.ds(off + k * 16, 16)] = zi
            bval[pl.ds(off + k * 16, 16)] = zf
            return 0

        lax.fori_loop(0, nz, _zf, 0)
        slotbase = (h * 32 + w) * SLOT
        pltpu.sync_copy(bsrc.at[pl.ds(0, SLOT)],
                        psrc.at[pl.ds(slotbase, SLOT)])
        pltpu.sync_copy(bdst.at[pl.ds(0, SLOT)],
                        pdst.at[pl.ds(slotbase, SLOT)])
        pltpu.sync_copy(bval.at[pl.ds(0, SLOT)],
                        pval.at[pl.ds(slotbase, SLOT)])
        cnt_v[pl.ds(0, 16)] = zi + pe
        pltpu.sync_copy(cnt_v, pcnt.at[pl.ds((h * 32 + w) * 16, 16)])


_part = functools.partial(
    pl.kernel,
    out_type=(
        jax.ShapeDtypeStruct((2 * 32 * SLOT,), _I32),
        jax.ShapeDtypeStruct((2 * 32 * SLOT,), _I32),
        jax.ShapeDtypeStruct((2 * 32 * SLOT,), _F32),
        jax.ShapeDtypeStruct((2 * 32 * 16,), _I32),
    ),
    mesh=_MESH,
    compiler_params=_PARAMS,
    scratch_types=[
        pltpu.VMEM((SB2,), _I32),
        pltpu.VMEM((SB2,), _I32),
        pltpu.VMEM((SB2,), _F32),
        pltpu.VMEM((BUFP,), _I32),
        pltpu.VMEM((BUFP,), _I32),
        pltpu.VMEM((BUFP,), _F32),
        pltpu.VMEM((16,), _I32),
    ],
)(_part_body)


def _scale_chunk(rows_v, ldst_v, sdst, sval, off0):
    for j in range(EC // 16):
        off = off0 + j * 16
        ldst_v[pl.ds(j * 16, 16)] = sdst[pl.ds(off, 16)]
        for e in range(16):
            bc = plsc.load_gather(sval, [jnp.full((16,), off + e, _I32)])
            row = j * 16 + e
            for d in range(4):
                sl = rows_v[row, pl.ds(d * 16, 16)]
                rows_v[row, pl.ds(d * 16, 16)] = sl * bc


def _layer_body(ecur, psrc, pdst, pval, pcnt, enext, acc,
                sidx, sdst, sval, cnt_v, ldst0, ldst1, ldst2, ldst3,
                rows0, rows1, rows2, rows3,
                stage, sem0, sem1, sem2, sem3):
    c = lax.axis_index("c")
    s = lax.axis_index("s")
    base_node = c * HALF
    zeros16 = jnp.zeros((16,), _F32)

    def _zrow(r, _):
        for d in range(4):
            stage[r, pl.ds(d * 16, 16)] = zeros16
        return 0

    lax.fori_loop(0, ZZ, _zrow, 0)
    tz = s * (ACC_ROWS // 16)

    def _zacc(i, _):
        pltpu.sync_copy(stage, acc.at[pl.ds(tz + i * ZZ, ZZ)])
        return 0

    with jax.named_scope("zerophase"):
        lax.fori_loop(0, ACC_ROWS // 16 // ZZ, _zacc, 0)
    plsc.subcore_barrier()

    def _gather(ch, rows_v, sem):
        pltpu.async_copy(ecur.at[sidx.at[pl.ds(ch * EC, EC)]], rows_v, sem)

    def _wait_g(rows_v, sem):
        with jax.named_scope("wg"):
            pltpu.make_async_copy(ecur.at[sidx.at[pl.ds(0, EC)]], rows_v,
                                  sem).wait()

    def _chunk(ch, rows_v, ldst_v):
        with jax.named_scope("comp"):
            _scale_chunk(rows_v, ldst_v, sdst, sval, ch * EC)
        with jax.named_scope("scat"):
            pltpu.sync_copy(rows_v, acc.at[ldst_v], add=True)

    rows_bufs = (rows0, rows1, rows2, rows3)
    ldst_bufs = (ldst0, ldst1, ldst2, ldst3)
    sems = (sem0, sem1, sem2, sem3)

    def _slot(q, _):
        p = 2 * s + q
        slotbase = (c * 32 + p) * SLOT
        pltpu.sync_copy(pcnt.at[pl.ds((c * 32 + p) * 16, 16)], cnt_v)
        cnt = jnp.sum(cnt_v[pl.ds(0, 16)]) >> 4
        nsb = cnt // SB

        def _sb(sb, _):
            base = slotbase + sb * SB
            with jax.named_scope("stage"):
                pltpu.sync_copy(psrc.at[pl.ds(base, SB)], sidx)
                pltpu.sync_copy(pdst.at[pl.ds(base, SB)], sdst)
                pltpu.sync_copy(pval.at[pl.ds(base, SB)], sval)
            with jax.named_scope("pipe"):
                for b in range(4):
                    _gather(b, rows_bufs[b], sems[b])

                def _round(m, _):
                    for b in range(4):
                        ch = 4 * m + b
                        _wait_g(rows_bufs[b], sems[b])
                        _chunk(ch, rows_bufs[b], ldst_bufs[b])

                        @pl.when(ch + 4 < CPS)
                        def _():
                            _gather(ch + 4, rows_bufs[b], sems[b])

                    return 0

                lax.fori_loop(0, CPS // 4, _round, 0)
            return 0

        lax.fori_loop(0, nsb, _sb, 0)
        return 0

    with jax.named_scope("edges"):
        lax.fori_loop(0, 2, _slot, 0)
    plsc.subcore_barrier()

    nch = jnp.where(s == 0, 40, 39)

    def _cp(k, _):
        r0 = (s + k * 16) * ZCH
        pltpu.sync_copy(acc.at[pl.ds(r0, ZCH)],
                        enext.at[pl.ds(base_node + r0, ZCH)])
        return 0

    lax.fori_loop(0, nch, _cp, 0)


_layer = functools.partial(
    pl.kernel,
    out_type=jax.ShapeDtypeStruct((N_NODES, DIM), _F32),
    mesh=_MESH,
    compiler_params=_PARAMS,
    scratch_types=[
        pltpu.VMEM_SHARED((ACC_ROWS, DIM), _F32),
        pltpu.VMEM((SB,), _I32),
        pltpu.VMEM((SB,), _I32),
        pltpu.VMEM((SB,), _F32),
        pltpu.VMEM((16,), _I32),
        pltpu.VMEM((EC,), _I32),
        pltpu.VMEM((EC,), _I32),
        pltpu.VMEM((EC,), _I32),
        pltpu.VMEM((EC,), _I32),
        pltpu.VMEM((EC, DIM), _F32),
        pltpu.VMEM((EC, DIM), _F32),
        pltpu.VMEM((EC, DIM), _F32),
        pltpu.VMEM((EC, DIM), _F32),
        pltpu.VMEM((ZZ, DIM), _F32),
        pltpu.SemaphoreType.DMA,
        pltpu.SemaphoreType.DMA,
        pltpu.SemaphoreType.DMA,
        pltpu.SemaphoreType.DMA,
    ],
)(_layer_body)


def _score_body(uidx, iidx, e0, l1, l2, l3, out, uv, iv, t0, t1, t2, t3,
                usum, isum, sc_v, sem):
    c = lax.axis_index("c")
    s = lax.axis_index("s")
    wid = s * 2 + c
    base = wid * PC
    pltpu.sync_copy(uidx.at[pl.ds(base, PC)], uv)
    pltpu.sync_copy(iidx.at[pl.ds(base, PC)], iv)

    def _gather4(idx_v, dst_sum):
        pltpu.async_copy(e0.at[idx_v], t0, sem).wait()
        pltpu.async_copy(l1.at[idx_v], t1, sem).wait()
        pltpu.async_copy(l2.at[idx_v], t2, sem).wait()
        pltpu.async_copy(l3.at[idx_v], t3, sem).wait()

        def _sumr(r, _):
            for d in range(4):
                sl = pl.ds(d * 16, 16)
                dst_sum[r, sl] = (t0[r, sl] + t1[r, sl] + t2[r, sl]
                                  + t3[r, sl])
            return 0

        lax.fori_loop(0, PC, _sumr, 0)

    _gather4(uv, usum)
    _gather4(iv, isum)

    lanes = jnp.arange(16, dtype=_I32)
    for g in range(PC // 16):
        rowsel = g * 16 + lanes
        acc = jnp.zeros((16,), _F32)
        for d in range(DIM):
            col = jnp.full((16,), d, _I32)
            ua = plsc.load_gather(usum, [rowsel, col])
            ia = plsc.load_gather(isum, [rowsel, col])
            acc = acc + ua * ia
        sc_v[pl.ds(g * 16, 16)] = acc * 0.0625

    pltpu.sync_copy(sc_v, out.at[pl.ds(base, PC)])


_score = functools.partial(
    pl.kernel,
    out_type=jax.ShapeDtypeStruct((BATCH,), _F32),
    mesh=_MESH,
    compiler_params=_PARAMS,
    scratch_types=[
        pltpu.VMEM((PC,), _I32),
        pltpu.VMEM((PC,), _I32),
        pltpu.VMEM((PC, DIM), _F32),
        pltpu.VMEM((PC, DIM), _F32),
        pltpu.VMEM((PC, DIM), _F32),
        pltpu.VMEM((PC, DIM), _F32),
        pltpu.VMEM((PC, DIM), _F32),
        pltpu.VMEM((PC, DIM), _F32),
        pltpu.VMEM((PC,), _F32),
        pltpu.SemaphoreType.DMA,
    ],
)(_score_body)


def kernel(users, items, adj_indices, adj_values, user_emb, item_emb):
    e0 = jnp.concatenate([user_emb, item_emb], axis=0)
    pad = E_PAD - N_EDGES
    src = jnp.pad(adj_indices[1].astype(_I32), (0, pad))
    dst = jnp.pad(adj_indices[0].astype(_I32), (0, pad))
    val = jnp.pad(adj_values, (0, pad))
    psrc, pdst, pval, pcnt = _part(src, dst, val)
    l1 = _layer(e0, psrc, pdst, pval, pcnt)
    l2 = _layer(l1, psrc, pdst, pval, pcnt)
    l3 = _layer(l2, psrc, pdst, pval, pcnt)
    return _score(users.astype(_I32), (items + N_USERS).astype(_I32),
                  e0, l1, l2, l3)
